# baseline (speedup 1.0000x reference)
import jax
import jax.numpy as jnp
from jax.experimental import pallas as pl

_DIM = 128
_OTHER_START = 1
_ROUTE_LEN = 1


def _make_body(seg_list, reps, total_len):
    def body(emb_ref, out_ref):
        e = emb_ref[...]
        for r in range(reps):
            base = r * total_len
            for (lo, ln, t) in seg_list:
                out_ref[base + lo:base + lo + ln, :] = jnp.broadcast_to(
                    e[t][None, :], (ln, _DIM))
    return body


def kernel(ego, obs, lane, bound, embedding):
    batch = ego.shape[0]
    obs_len = obs.shape[1]
    lanes_len = lane.shape[1]
    bounds_len = bound.shape[1]
    total_len = 1 + obs_len + _ROUTE_LEN + lanes_len + bounds_len

    route_start = _OTHER_START + obs_len
    lanes_start = route_start + _ROUTE_LEN
    bounds_start = lanes_start + lanes_len
    segs = [
        (0, 1, 0),
        (_OTHER_START, obs_len, 2),
        (route_start, _ROUTE_LEN, 3),
        (lanes_start, lanes_len, 4),
        (bounds_start, bounds_len, 5),
    ]

    reps = 4
    rows_per_block = reps * total_len
    total_rows = batch * total_len
    grid = (total_rows // rows_per_block,)

    out = pl.pallas_call(
        _make_body(segs, reps, total_len),
        grid=grid,
        in_specs=[pl.BlockSpec((embedding.shape[0], _DIM), lambda i: (0, 0))],
        out_specs=pl.BlockSpec((rows_per_block, _DIM), lambda i: (i, 0)),
        out_shape=jax.ShapeDtypeStruct((total_rows, _DIM), embedding.dtype),
    )(embedding)
    return out.reshape(batch, total_len, _DIM)


# speedup vs baseline: 2.9998x; 2.9998x over previous
import jax
import jax.numpy as jnp
from jax.experimental import pallas as pl
from jax.experimental.pallas import tpu as pltpu

_DIM = 128
_OTHER_START = 1
_ROUTE_LEN = 1


def _make_body(seg_list, bb):
    def body(emb_ref, out_ref):
        e = emb_ref[...]
        for (lo, ln, t) in seg_list:
            out_ref[:, lo:lo + ln, :] = jnp.broadcast_to(
                e[t][None, None, :], (bb, ln, _DIM))
    return body


def kernel(ego, obs, lane, bound, embedding):
    batch = ego.shape[0]
    obs_len = obs.shape[1]
    lanes_len = lane.shape[1]
    bounds_len = bound.shape[1]
    total_len = 1 + obs_len + _ROUTE_LEN + lanes_len + bounds_len

    route_start = _OTHER_START + obs_len
    lanes_start = route_start + _ROUTE_LEN
    bounds_start = lanes_start + lanes_len
    segs = [
        (0, 1, 0),
        (_OTHER_START, obs_len, 2),
        (route_start, _ROUTE_LEN, 3),
        (lanes_start, lanes_len, 4),
        (bounds_start, bounds_len, 5),
    ]

    bb = 64
    while batch % bb != 0:
        bb //= 2
    grid = (batch // bb,)

    out = pl.pallas_call(
        _make_body(segs, bb),
        grid=grid,
        in_specs=[pl.BlockSpec((embedding.shape[0], _DIM), lambda i: (0, 0))],
        out_specs=pl.BlockSpec((bb, total_len, _DIM), lambda i: (i, 0, 0)),
        out_shape=jax.ShapeDtypeStruct((batch, total_len, _DIM),
                                       embedding.dtype),
        compiler_params=pltpu.CompilerParams(
            dimension_semantics=("parallel",)),
    )(embedding)
    return out


# speedup vs baseline: 3.0092x; 1.0031x over previous
import jax
import jax.numpy as jnp
from jax.experimental import pallas as pl
from jax.experimental.pallas import tpu as pltpu

_DIM = 128
_OTHER_START = 1
_ROUTE_LEN = 1


def _make_body(seg_list, batch, group, n_sems):
    num_copies = batch // group

    def body(emb_ref, out_hbm, pat_ref, sems):
        e = emb_ref[...]
        for (lo, ln, t) in seg_list:
            pat_ref[:, lo:lo + ln, :] = jnp.broadcast_to(
                e[t][None, None, :], (group, ln, _DIM))
        copies = [
            pltpu.make_async_copy(
                pat_ref,
                out_hbm.at[pl.ds(i * group, group)],
                sems.at[i % n_sems],
            )
            for i in range(num_copies)
        ]
        for i, cp in enumerate(copies):
            if i >= n_sems:
                copies[i - n_sems].wait()
            cp.start()
        for cp in copies[-n_sems:]:
            cp.wait()

    return body


def kernel(ego, obs, lane, bound, embedding):
    batch = ego.shape[0]
    obs_len = obs.shape[1]
    lanes_len = lane.shape[1]
    bounds_len = bound.shape[1]
    total_len = 1 + obs_len + _ROUTE_LEN + lanes_len + bounds_len

    route_start = _OTHER_START + obs_len
    lanes_start = route_start + _ROUTE_LEN
    bounds_start = lanes_start + lanes_len
    segs = [
        (0, 1, 0),
        (_OTHER_START, obs_len, 2),
        (route_start, _ROUTE_LEN, 3),
        (lanes_start, lanes_len, 4),
        (bounds_start, bounds_len, 5),
    ]

    group = 16
    while batch % group != 0:
        group //= 2
    n_sems = 8

    out = pl.pallas_call(
        _make_body(segs, batch, group, n_sems),
        in_specs=[pl.BlockSpec(memory_space=pltpu.VMEM)],
        out_specs=pl.BlockSpec(memory_space=pl.ANY),
        out_shape=jax.ShapeDtypeStruct((batch, total_len, _DIM),
                                       embedding.dtype),
        scratch_shapes=[
            pltpu.VMEM((group, total_len, _DIM), embedding.dtype),
            pltpu.SemaphoreType.DMA((n_sems,)),
        ],
    )(embedding)
    return out
